# baseline (speedup 1.0000x reference)
import functools

import jax
import jax.numpy as jnp
from jax import lax
from jax.experimental import pallas as pl
from jax.experimental.pallas import tpu as pltpu
from jax.experimental.pallas import tpu_sc as plsc

VOCAB = 1000000
EMBED = 64
BATCH = 16384

NC = 2
NS = 16
NW = NC * NS
BPW = BATCH // NW
CHUNK = 128
NCH = BPW // CHUNK
GPC = CHUNK // 16

_LN2 = 0.6931471805599453
_SQRT2 = 1.4142135623730951


def _ln(x):
    bits = plsc.bitcast(x, jnp.int32)
    e = lax.shift_right_arithmetic(bits, 23) - 127
    m = plsc.bitcast((bits & 0x7FFFFF) | 0x3F800000, jnp.float32)
    big = m > _SQRT2
    m = jnp.where(big, m * 0.5, m)
    ef = (e + jnp.where(big, 1, 0)).astype(jnp.float32)
    t = (m - 1.0) / (m + 1.0)
    t2 = t * t
    poly = 2.0 * t * (1.0 + t2 * (1.0 / 3.0 + t2 * (0.2 + t2 * (1.0 / 7.0))))
    return poly + ef * _LN2


def _glove_body(row_hbm, col_hbm, x_hbm, p_hbm, ce_hbm, bet_hbm, cb_hbm,
                bb_hbm, out_hbm, idx_r, idx_c, v_buf, u_buf, cb_v, bb_v,
                x_v, p_v, out_v, sem0, sem1, sem2, sem3, sem_u):
    wid = lax.axis_index("s") * NC + lax.axis_index("c")
    base = pl.multiple_of(wid * BPW, BPW)

    pltpu.sync_copy(row_hbm.at[pl.ds(wid * NCH, NCH)], idx_r)
    pltpu.sync_copy(col_hbm.at[pl.ds(base, BPW)], idx_c)
    pltpu.sync_copy(x_hbm.at[pl.ds(base, BPW)], x_v)
    pltpu.sync_copy(p_hbm.at[pl.ds(base, BPW)], p_v)

    u_waits = [pltpu.async_copy(bb_hbm.at[idx_c], bb_v, sem_u)]
    for e in range(EMBED):
        u_waits.append(
            pltpu.async_copy(bet_hbm.at[e].at[idx_c], u_buf.at[e], sem_u))

    sems = [sem0, sem1, sem2, sem3]
    waits = []
    for j in range(NCH):
        dst = pl.ds(j * CHUNK, CHUNK)
        waits.append((
            pltpu.async_copy(ce_hbm.at[idx_r.at[j]], v_buf.at[dst], sems[j]),
            pltpu.async_copy(cb_hbm.at[idx_r.at[j]], cb_v.at[dst], sems[j]),
        ))

    for w in u_waits:
        w.wait()

    iota = lax.iota(jnp.int32, 16)
    ones = jnp.ones((16,), jnp.int32)

    for j in range(NCH):
        for d in waits[j]:
            d.wait()

        def g_body(g, carry, j=j):
            off = pl.multiple_of(j * CHUNK + g * 16, 16)
            row_idx = off + iota
            acc = jnp.zeros((16,), jnp.float32)
            col = jnp.zeros((16,), jnp.int32)
            for e in range(EMBED):
                v = plsc.load_gather(v_buf, [row_idx, col])
                acc = acc + v * u_buf[e, pl.ds(off, 16)]
                if e + 1 < EMBED:
                    col = col + ones
            d = acc + cb_v[pl.ds(off, 16)] + bb_v[pl.ds(off, 16)] \
                - _ln(x_v[pl.ds(off, 16)])
            out_v[pl.ds(off, 16)] = p_v[pl.ds(off, 16)] * d * d
            return carry

        lax.fori_loop(0, GPC, g_body, None)

    pltpu.sync_copy(out_v, out_hbm.at[pl.ds(base, BPW)])


@jax.jit
def _glove(row2, col, x, p, ce, bet, cb, bb):
    mesh = plsc.VectorSubcoreMesh(core_axis_name="c", subcore_axis_name="s")
    run = pl.kernel(
        _glove_body,
        out_type=jax.ShapeDtypeStruct((BATCH,), jnp.float32),
        mesh=mesh,
        compiler_params=pltpu.CompilerParams(
            needs_layout_passes=False, use_tc_tiling_on_sc=False),
        scratch_types=[
            pltpu.VMEM((NCH, CHUNK), jnp.int32),
            pltpu.VMEM((BPW,), jnp.int32),
            pltpu.VMEM((BPW, EMBED), jnp.float32),
            pltpu.VMEM((EMBED, BPW), jnp.float32),
            pltpu.VMEM((BPW,), jnp.float32),
            pltpu.VMEM((BPW,), jnp.float32),
            pltpu.VMEM((BPW,), jnp.float32),
            pltpu.VMEM((BPW,), jnp.float32),
            pltpu.VMEM((BPW,), jnp.float32),
            pltpu.SemaphoreType.DMA,
            pltpu.SemaphoreType.DMA,
            pltpu.SemaphoreType.DMA,
            pltpu.SemaphoreType.DMA,
            pltpu.SemaphoreType.DMA,
        ],
    )
    return run(row2, col, x, p, ce, bet, cb, bb)


def kernel(row, column, x_ik, punish_x, center_embed, background_embed,
           center_bias, background_bias):
    row2 = row.reshape(NW * NCH, CHUNK)
    x = x_ik.reshape(BATCH)
    p = punish_x.reshape(BATCH)
    bet = background_embed.T
    cb = center_bias.reshape(VOCAB)
    bb = background_bias.reshape(VOCAB)
    return _glove(row2, column, x, p, center_embed, bet, cb, bb)


# speedup vs baseline: 4.9134x; 4.9134x over previous
import functools

import jax
import jax.numpy as jnp
from jax import lax
from jax.experimental import pallas as pl
from jax.experimental.pallas import tpu as pltpu
from jax.experimental.pallas import tpu_sc as plsc

VOCAB = 1000000
EMBED = 64
BATCH = 16384

NC = 2
NS = 16
NW = NC * NS
BPW = BATCH // NW
CHUNK = 128
NCH = BPW // CHUNK
GPC = CHUNK // 16

_LN2 = 0.6931471805599453
_SQRT2 = 1.4142135623730951


def _ln(x):
    bits = plsc.bitcast(x, jnp.int32)
    e = lax.shift_right_arithmetic(bits, 23) - 127
    m = plsc.bitcast((bits & 0x7FFFFF) | 0x3F800000, jnp.float32)
    big = m > _SQRT2
    m = jnp.where(big, m * 0.5, m)
    ef = (e + jnp.where(big, 1, 0)).astype(jnp.float32)
    t = (m - 1.0) / (m + 1.0)
    t2 = t * t
    poly = 2.0 * t * (1.0 + t2 * (1.0 / 3.0 + t2 * (0.2 + t2 * (1.0 / 7.0))))
    return poly + ef * _LN2


def _glove_body(row_hbm, col_hbm, x_hbm, p_hbm, ce_hbm, be_hbm, cb_hbm,
                bb_hbm, out_hbm, idx_r, idx_c, v_buf, u_buf, cb_v, bb_v,
                x_v, p_v, out_v, sem0, sem1, sem2, sem3):
    wid = lax.axis_index("s") * NC + lax.axis_index("c")
    base = pl.multiple_of(wid * BPW, BPW)

    pltpu.sync_copy(row_hbm.at[pl.ds(wid * NCH, NCH)], idx_r)
    pltpu.sync_copy(col_hbm.at[pl.ds(wid * NCH, NCH)], idx_c)
    pltpu.sync_copy(x_hbm.at[pl.ds(base, BPW)], x_v)
    pltpu.sync_copy(p_hbm.at[pl.ds(base, BPW)], p_v)

    sems = [sem0, sem1, sem2, sem3]
    waits = []
    for j in range(NCH):
        s = sems[j]
        dst = pl.ds(j * CHUNK, CHUNK)
        waits.append((
            pltpu.async_copy(ce_hbm.at[idx_r.at[j]], v_buf.at[dst], s),
            pltpu.async_copy(be_hbm.at[idx_c.at[j]], u_buf.at[dst], s),
            pltpu.async_copy(cb_hbm.at[idx_r.at[j]], cb_v.at[dst], s),
            pltpu.async_copy(bb_hbm.at[idx_c.at[j]], bb_v.at[dst], s),
        ))

    iota = lax.iota(jnp.int32, 16)
    ones = jnp.ones((16,), jnp.int32)

    for j in range(NCH):
        for d in waits[j]:
            d.wait()

        def g_body(g, carry, j=j):
            off = pl.multiple_of(j * CHUNK + g * 16, 16)
            row_idx = off + iota
            acc = jnp.zeros((16,), jnp.float32)
            col = jnp.zeros((16,), jnp.int32)
            for e in range(EMBED):
                v = plsc.load_gather(v_buf, [row_idx, col])
                u = plsc.load_gather(u_buf, [row_idx, col])
                acc = acc + v * u
                if e + 1 < EMBED:
                    col = col + ones
            d = acc + cb_v[pl.ds(off, 16)] + bb_v[pl.ds(off, 16)] \
                - _ln(x_v[pl.ds(off, 16)])
            out_v[pl.ds(off, 16)] = p_v[pl.ds(off, 16)] * d * d
            return carry

        lax.fori_loop(0, GPC, g_body, None)

    pltpu.sync_copy(out_v, out_hbm.at[pl.ds(base, BPW)])


@jax.jit
def _glove(row2, col2, x, p, ce, be, cb, bb):
    mesh = plsc.VectorSubcoreMesh(core_axis_name="c", subcore_axis_name="s")
    run = pl.kernel(
        _glove_body,
        out_type=jax.ShapeDtypeStruct((BATCH,), jnp.float32),
        mesh=mesh,
        compiler_params=pltpu.CompilerParams(
            needs_layout_passes=False, use_tc_tiling_on_sc=False),
        scratch_types=[
            pltpu.VMEM((NCH, CHUNK), jnp.int32),
            pltpu.VMEM((NCH, CHUNK), jnp.int32),
            pltpu.VMEM((BPW, EMBED), jnp.float32),
            pltpu.VMEM((BPW, EMBED), jnp.float32),
            pltpu.VMEM((BPW,), jnp.float32),
            pltpu.VMEM((BPW,), jnp.float32),
            pltpu.VMEM((BPW,), jnp.float32),
            pltpu.VMEM((BPW,), jnp.float32),
            pltpu.VMEM((BPW,), jnp.float32),
            pltpu.SemaphoreType.DMA,
            pltpu.SemaphoreType.DMA,
            pltpu.SemaphoreType.DMA,
            pltpu.SemaphoreType.DMA,
        ],
    )
    return run(row2, col2, x, p, ce, be, cb, bb)


def kernel(row, column, x_ik, punish_x, center_embed, background_embed,
           center_bias, background_bias):
    row2 = row.reshape(NW * NCH, CHUNK)
    col2 = column.reshape(NW * NCH, CHUNK)
    x = x_ik.reshape(BATCH)
    p = punish_x.reshape(BATCH)
    cb = center_bias.reshape(VOCAB)
    bb = background_bias.reshape(VOCAB)
    return _glove(row2, col2, x, p, center_embed, background_embed, cb, bb)
